# baseline (speedup 1.0000x reference)
import functools

import jax
import jax.numpy as jnp
from jax import lax
from jax.experimental import pallas as pl
from jax.experimental.pallas import tpu as pltpu, tpu_sc as plsc

N_IN = 200000
N_OUT = 50000
D = 256
L = 16
S = 125
NCH = N_OUT // S
R = 128
NC = 2
NS = 16
NW = NC * NS
OFF_PAD = 416


def _sc_body(feat_hbm, ids_hbm, off_hbm, out_hbm, xbuf, accbuf, offs_sm, idbuf_sm):
    wid = lax.axis_index("s") * NC + lax.axis_index("c")
    pltpu.sync_copy(off_hbm, offs_sm)
    neg = jnp.full((L,), -jnp.inf, dtype=jnp.float32)

    def chunk_body(i, carry):
        c = i * NW + wid

        @pl.when(c < NCH)
        def _():
            base = c * S
            offv = offs_sm[pl.ds(c, L)]
            r_lo = offv[0]
            r_hi = offv[1]

            def initb(k, carry2):
                ko = k * (8 * L)
                for u in range(8):
                    accbuf[pl.ds(ko + u * L, L)] = neg
                return carry2

            lax.fori_loop(0, S * D // (8 * L), initb, 0)

            start0 = (r_lo // 8) * 8
            nsteps = (r_hi - start0 + R - 1) // R

            def step_body(s, carry2):
                st = jnp.minimum(start0 + s * R, N_IN - R)
                pltpu.sync_copy(feat_hbm.at[pl.ds(st * D, R * D)], xbuf)
                pltpu.sync_copy(ids_hbm.at[pl.ds(st, R)], idbuf_sm)

                def grp_body(g, carry3):
                    idvec = idbuf_sm[pl.ds(g * L, L)]
                    for jj in range(L):
                        sid = idvec[jj]

                        @pl.when((sid >= base) & (sid < base + S))
                        def _(sid=sid, jj=jj, g=g):
                            ao = (sid - base) * D
                            xo = (g * L + jj) * D
                            for j in range(D // L):
                                accbuf[pl.ds(ao + j * L, L)] = jnp.maximum(
                                    accbuf[pl.ds(ao + j * L, L)],
                                    xbuf[pl.ds(xo + j * L, L)],
                                )

                    return carry3

                lax.fori_loop(0, R // L, grp_body, 0)
                return carry2

            lax.fori_loop(0, nsteps, step_body, 0)
            pltpu.sync_copy(accbuf, out_hbm.at[pl.ds(base * D, S * D)])

        return carry

    lax.fori_loop(0, (NCH + NW - 1) // NW, chunk_body, 0)


@jax.jit
def kernel(features, out_site_ids):
    bounds = jnp.arange(NCH + 1, dtype=jnp.int32) * S
    offs = jnp.searchsorted(out_site_ids, bounds, side="left").astype(jnp.int32)
    offs = jnp.concatenate(
        [offs, jnp.full((OFF_PAD - NCH - 1,), N_IN, dtype=jnp.int32)]
    )
    feat_flat = features.reshape(-1)

    mesh = plsc.VectorSubcoreMesh(core_axis_name="c", subcore_axis_name="s")
    out_flat = pl.kernel(
        _sc_body,
        out_type=jax.ShapeDtypeStruct((N_OUT * D,), jnp.float32),
        mesh=mesh,
        scratch_types=[
            pltpu.VMEM((R * D,), jnp.float32),
            pltpu.VMEM((S * D,), jnp.float32),
            pltpu.VMEM((OFF_PAD,), jnp.int32),
            pltpu.VMEM((R,), jnp.int32),
        ],
    )(feat_flat, out_site_ids, offs)
    return out_flat.reshape(N_OUT, D)


# speedup vs baseline: 1.1242x; 1.1242x over previous
import functools

import jax
import jax.numpy as jnp
from jax import lax
from jax.experimental import pallas as pl
from jax.experimental.pallas import tpu as pltpu, tpu_sc as plsc

N_IN = 200000
N_OUT = 50000
D = 256
L = 16
S = 125
NCH = N_OUT // S
R = 128
NC = 2
NS = 16
NW = NC * NS
OFF_PAD = 416


def _sc_body(feat_hbm, ids_hbm, off_hbm, out_hbm, xbuf, accbuf, offs_sm, idbuf_sm, sem):
    wid = lax.axis_index("s") * NC + lax.axis_index("c")
    pltpu.sync_copy(off_hbm, offs_sm)
    neg = jnp.full((L,), -jnp.inf, dtype=jnp.float32)

    def chunk_body(i, carry):
        c = i * NW + wid

        @pl.when(c < NCH)
        def _():
            base = c * S
            offv = offs_sm[pl.ds(c, L)]
            r_lo = offv[0]
            r_hi = offv[1]

            def initb(k, carry2):
                ko = k * (8 * L)
                for u in range(8):
                    accbuf[pl.ds(ko + u * L, L)] = neg
                return carry2

            lax.fori_loop(0, S * D // (8 * L), initb, 0)

            start0 = (r_lo // 8) * 8
            nsteps = (r_hi - start0 + R - 1) // R

            def issue(s):
                st = jnp.minimum(start0 + s * R, N_IN - R)
                b = lax.rem(s, 2)
                pltpu.make_async_copy(
                    feat_hbm.at[pl.ds(st * D, R * D)],
                    xbuf.at[pl.ds(b * R * D, R * D)],
                    sem.at[b],
                ).start()
                pltpu.make_async_copy(
                    ids_hbm.at[pl.ds(st, R)],
                    idbuf_sm.at[pl.ds(b * R, R)],
                    sem.at[b],
                ).start()

            def drain(s):
                st = jnp.minimum(start0 + s * R, N_IN - R)
                b = lax.rem(s, 2)
                pltpu.make_async_copy(
                    feat_hbm.at[pl.ds(st * D, R * D)],
                    xbuf.at[pl.ds(b * R * D, R * D)],
                    sem.at[b],
                ).wait()
                pltpu.make_async_copy(
                    ids_hbm.at[pl.ds(st, R)],
                    idbuf_sm.at[pl.ds(b * R, R)],
                    sem.at[b],
                ).wait()

            issue(0)

            def step_body(s, carry2):
                drain(s)

                @pl.when(s + 1 < nsteps)
                def _():
                    issue(s + 1)

                b = lax.rem(s, 2)
                xb = b * R * D

                def grp_body(g, carry3):
                    idvec = idbuf_sm[pl.ds(b * R + g * L, L)]
                    for jj in range(L):
                        sid = idvec[jj]

                        @pl.when((sid >= base) & (sid < base + S))
                        def _(sid=sid, jj=jj, g=g):
                            ao = (sid - base) * D
                            xo = xb + (g * L + jj) * D
                            for j in range(D // L):
                                accbuf[pl.ds(ao + j * L, L)] = jnp.maximum(
                                    accbuf[pl.ds(ao + j * L, L)],
                                    xbuf[pl.ds(xo + j * L, L)],
                                )

                    return carry3

                lax.fori_loop(0, R // L, grp_body, 0)
                return carry2

            lax.fori_loop(0, nsteps, step_body, 0)
            pltpu.sync_copy(accbuf, out_hbm.at[pl.ds(base * D, S * D)])

        return carry

    lax.fori_loop(0, (NCH + NW - 1) // NW, chunk_body, 0)


@jax.jit
def kernel(features, out_site_ids):
    bounds = jnp.arange(NCH + 1, dtype=jnp.int32) * S
    offs = jnp.searchsorted(out_site_ids, bounds, side="left").astype(jnp.int32)
    offs = jnp.concatenate(
        [offs, jnp.full((OFF_PAD - NCH - 1,), N_IN, dtype=jnp.int32)]
    )
    feat_flat = features.reshape(-1)

    mesh = plsc.VectorSubcoreMesh(core_axis_name="c", subcore_axis_name="s")
    out_flat = pl.kernel(
        _sc_body,
        out_type=jax.ShapeDtypeStruct((N_OUT * D,), jnp.float32),
        mesh=mesh,
        scratch_types=[
            pltpu.VMEM((2 * R * D,), jnp.float32),
            pltpu.VMEM((S * D,), jnp.float32),
            pltpu.VMEM((OFF_PAD,), jnp.int32),
            pltpu.VMEM((2 * R,), jnp.int32),
            pltpu.SemaphoreType.DMA((2,)),
        ],
    )(feat_flat, out_site_ids, offs)
    return out_flat.reshape(N_OUT, D)


# speedup vs baseline: 2.3357x; 2.0776x over previous
import jax
import jax.numpy as jnp
from jax import lax
from jax.experimental import pallas as pl
from jax.experimental.pallas import tpu as pltpu, tpu_sc as plsc

N_IN = 200000
N_OUT = 50000
D = 256
L = 16
NJ = D // L
S = 125
NCH = N_OUT // S
R = 128
NC = 2
NS = 16
NW = NC * NS
OFF_PAD = 416


def _sc_body(feat_hbm, ids_hbm, off_hbm, out_hbm, xbuf, accbuf, offs_vm, idbuf, sem):
    wid = lax.axis_index("s") * NC + lax.axis_index("c")
    pltpu.sync_copy(off_hbm, offs_vm)
    neg = jnp.full((L,), -jnp.inf, dtype=jnp.float32)

    def chunk_body(i, carry):
        c = i * NW + wid

        @pl.when(c < NCH)
        def _():
            base = c * S
            offv = offs_vm[pl.ds(c, L)]
            r_lo = offv[0]
            r_hi = offv[1]
            start0 = (r_lo // 8) * 8
            nsteps = (r_hi - start0 + R - 1) // R

            def issue(s):
                st = jnp.minimum(start0 + s * R, N_IN - R)
                b = lax.rem(s, 2)
                pltpu.make_async_copy(
                    feat_hbm.at[pl.ds(st * D, R * D)],
                    xbuf.at[pl.ds(b * R * D, R * D)],
                    sem.at[b],
                ).start()
                pltpu.make_async_copy(
                    ids_hbm.at[pl.ds(st, R)],
                    idbuf.at[pl.ds(b * R, R)],
                    sem.at[b],
                ).start()

            def drain(s):
                st = jnp.minimum(start0 + s * R, N_IN - R)
                b = lax.rem(s, 2)
                pltpu.make_async_copy(
                    feat_hbm.at[pl.ds(st * D, R * D)],
                    xbuf.at[pl.ds(b * R * D, R * D)],
                    sem.at[b],
                ).wait()
                pltpu.make_async_copy(
                    ids_hbm.at[pl.ds(st, R)],
                    idbuf.at[pl.ds(b * R, R)],
                    sem.at[b],
                ).wait()

            issue(0)

            def step_body(s, carry2):
                drain(s)
                b = lax.rem(s, 2)

                @pl.when(s + 1 < nsteps)
                def _():
                    issue(s + 1)

                st = jnp.minimum(start0 + s * R, N_IN - R)
                lo = start0 + s * R - st
                xb = b * R * D

                def grp_body(g, carry3):
                    cur = carry3[0]
                    acc = list(carry3[1:])
                    idvec = idbuf[pl.ds(b * R + g * L, L)]
                    for jj in range(L):
                        r = g * L + jj
                        sid = idvec[jj]
                        xo = xb + r * D
                        xs = [xbuf[pl.ds(xo + j * L, L)] for j in range(NJ)]
                        proc = (r >= lo) & (sid >= base) & (sid < base + S)
                        reinit = proc & (sid != cur)
                        ao = jnp.clip(cur - base, 0, S) * D
                        for j in range(NJ):
                            accbuf[pl.ds(ao + j * L, L)] = acc[j]
                        cur = jnp.where(proc, sid, cur)
                        acc = [
                            jnp.where(
                                reinit,
                                xs[j],
                                jnp.where(proc, jnp.maximum(acc[j], xs[j]), acc[j]),
                            )
                            for j in range(NJ)
                        ]
                    return (cur,) + tuple(acc)

                return lax.fori_loop(0, R // L, grp_body, carry2)

            init = (base + S,) + tuple(neg for _ in range(NJ))
            fin = lax.fori_loop(0, nsteps, step_body, init)
            cur = fin[0]
            floc = jnp.clip(cur - base, 0, S)
            ao = floc * D
            for j in range(NJ):
                accbuf[pl.ds(ao + j * L, L)] = fin[1 + j]
            pltpu.sync_copy(
                accbuf.at[pl.ds(0, S * D)], out_hbm.at[pl.ds(base * D, S * D)]
            )

        return carry

    lax.fori_loop(0, (NCH + NW - 1) // NW, chunk_body, 0)


@jax.jit
def kernel(features, out_site_ids):
    bounds = jnp.arange(NCH + 1, dtype=jnp.int32) * S
    offs = jnp.searchsorted(out_site_ids, bounds, side="left").astype(jnp.int32)
    offs = jnp.concatenate(
        [offs, jnp.full((OFF_PAD - NCH - 1,), N_IN, dtype=jnp.int32)]
    )
    feat_flat = features.reshape(-1)

    mesh = plsc.VectorSubcoreMesh(core_axis_name="c", subcore_axis_name="s")
    out_flat = pl.kernel(
        _sc_body,
        out_type=jax.ShapeDtypeStruct((N_OUT * D,), jnp.float32),
        mesh=mesh,
        scratch_types=[
            pltpu.VMEM((2 * R * D,), jnp.float32),
            pltpu.VMEM(((S + 1) * D,), jnp.float32),
            pltpu.VMEM((OFF_PAD,), jnp.int32),
            pltpu.VMEM((2 * R,), jnp.int32),
            pltpu.SemaphoreType.DMA((2,)),
        ],
    )(feat_flat, out_site_ids, offs)
    return out_flat.reshape(N_OUT, D)
